# Initial kernel scaffold; baseline (speedup 1.0000x reference)
import jax
import jax.numpy as jnp
from jax.experimental import pallas as pl


def kernel(targets, contexts, negatives, input_w, output_w):
    raise NotImplementedError("write your pallas kernel here")


# speedup vs baseline: 2.7925x; 2.7925x over previous
import functools

import jax
import jax.numpy as jnp
from jax import lax
from jax.experimental import pallas as pl
from jax.experimental.pallas import tpu as pltpu
from jax.experimental.pallas import tpu_sc as plsc

_NC = 2
_NS = 16
_L = 16
_NW = _NC * _NS
_CHUNK = 64


def _sc_body(K, D, t_hbm, c_hbm, n_hbm, iw_hbm, ow_hbm, pos_out, neg_out,
             t_idx, c_idx, n_idx, v_rows, up_rows, un_rows, pos_part,
             neg_part, sem):
    B = t_hbm.shape[0]
    per_w = B // _NW
    n_chunks = per_w // _CHUNK
    nd = D // _L
    wid = lax.axis_index("s") * _NC + lax.axis_index("c")

    def chunk(it, carry):
        base = wid * per_w + it * _CHUNK
        pltpu.sync_copy(t_hbm.at[pl.ds(base, _CHUNK)], t_idx)
        pltpu.sync_copy(c_hbm.at[pl.ds(base, _CHUNK)], c_idx)
        for k in range(K):
            pltpu.sync_copy(n_hbm.at[k, pl.ds(base, _CHUNK)], n_idx.at[k])
        cps = [
            pltpu.async_copy(iw_hbm.at[t_idx], v_rows, sem),
            pltpu.async_copy(ow_hbm.at[c_idx], up_rows, sem),
        ]
        for k in range(K):
            cps.append(pltpu.async_copy(ow_hbm.at[n_idx.at[k]], un_rows.at[k], sem))
        for cp in cps:
            cp.wait()

        def elem(b, carry2):
            v = [v_rows[b, pl.ds(j * _L, _L)] for j in range(nd)]
            u = [up_rows[b, pl.ds(j * _L, _L)] for j in range(nd)]
            acc = v[0] * u[0]
            for j in range(1, nd):
                acc = acc + v[j] * u[j]
            pos_part[b, :] = acc
            for k in range(K):
                w = [un_rows[k, b, pl.ds(j * _L, _L)] for j in range(nd)]
                nacc = v[0] * w[0]
                for j in range(1, nd):
                    nacc = nacc + v[j] * w[j]
                neg_part[k, b, :] = nacc
            return carry2

        lax.fori_loop(0, _CHUNK, elem, 0)
        pltpu.sync_copy(pos_part, pos_out.at[pl.ds(base, _CHUNK)])
        for k in range(K):
            pltpu.sync_copy(neg_part.at[k], neg_out.at[k, pl.ds(base, _CHUNK)])
        return carry

    lax.fori_loop(0, n_chunks, chunk, 0)


@functools.lru_cache(maxsize=None)
def _make_sc(B, K, D):
    mesh = plsc.VectorSubcoreMesh(core_axis_name="c", subcore_axis_name="s")
    return pl.kernel(
        functools.partial(_sc_body, K, D),
        out_type=[
            jax.ShapeDtypeStruct((B, _L), jnp.float32),
            jax.ShapeDtypeStruct((K, B, _L), jnp.float32),
        ],
        mesh=mesh,
        compiler_params=pltpu.CompilerParams(use_tc_tiling_on_sc=False),
        scratch_types=[
            pltpu.VMEM((_CHUNK,), jnp.int32),
            pltpu.VMEM((_CHUNK,), jnp.int32),
            pltpu.VMEM((K, _CHUNK), jnp.int32),
            pltpu.VMEM((_CHUNK, D), jnp.float32),
            pltpu.VMEM((_CHUNK, D), jnp.float32),
            pltpu.VMEM((K, _CHUNK, D), jnp.float32),
            pltpu.VMEM((_CHUNK, _L), jnp.float32),
            pltpu.VMEM((K, _CHUNK, _L), jnp.float32),
            pltpu.SemaphoreType.DMA,
        ],
    )


def _loss_body(B, pos_ref, neg_ref, out_ref):
    i = pl.program_id(0)

    @pl.when(i == 0)
    def _init():
        out_ref[0, 0] = 0.0

    col = lax.broadcasted_iota(jnp.int32, (128, 128), 0) // _L
    row = lax.broadcasted_iota(jnp.int32, (128, 128), 1)
    sel = jnp.where((col == row) & (row < 128 // _L), 1.0, 0.0)
    valid = lax.broadcasted_iota(jnp.int32, (1, 128), 1) < 128 // _L

    def logsig(x):
        return jnp.minimum(x, 0.0) - jnp.log1p(jnp.exp(-jnp.abs(x)))

    ps = jnp.dot(pos_ref[:], sel, preferred_element_type=jnp.float32)
    ns = jnp.dot(neg_ref[:], sel, preferred_element_type=jnp.float32)
    tot = (jnp.sum(jnp.where(valid, logsig(ps), 0.0))
           + jnp.sum(jnp.where(valid, logsig(-ns), 0.0)))
    out_ref[0, 0] += -tot / B


def _tc_loss(pos2d, neg2d, B):
    n_steps = 8
    pb = pos2d.shape[0] // n_steps
    nb = neg2d.shape[0] // n_steps
    out = pl.pallas_call(
        functools.partial(_loss_body, B),
        grid=(n_steps,),
        in_specs=[
            pl.BlockSpec((pb, 128), lambda i: (i, 0)),
            pl.BlockSpec((nb, 128), lambda i: (i, 0)),
        ],
        out_specs=pl.BlockSpec(memory_space=pltpu.SMEM),
        out_shape=jax.ShapeDtypeStruct((1, 1), jnp.float32),
    )(pos2d, neg2d)
    return out[0, 0]


def kernel(targets, contexts, negatives, input_w, output_w):
    B = targets.shape[0]
    K = negatives.shape[1]
    D = input_w.shape[1]
    neg_t = jnp.transpose(negatives)
    pos_part, neg_part = _make_sc(B, K, D)(
        targets, contexts, neg_t, input_w, output_w)
    pos2d = pos_part.reshape(-1, 128)
    neg2d = neg_part.reshape(-1, 128)
    return _tc_loss(pos2d, neg2d, B)


# speedup vs baseline: 2.8657x; 1.0262x over previous
import functools

import jax
import jax.numpy as jnp
from jax import lax
from jax.experimental import pallas as pl
from jax.experimental.pallas import tpu as pltpu
from jax.experimental.pallas import tpu_sc as plsc

_NC = 2
_NS = 16
_L = 16
_NW = _NC * _NS
_CHUNK = 64


def _sc_body(K, D, t_hbm, c_hbm, n_hbm, iw_hbm, ow_hbm, pos_out, neg_out,
             t_idx, c_idx, n_idx, v_rows, up_rows, un_rows, pos_part,
             neg_part, sem):
    B = t_hbm.shape[0]
    per_w = B // _NW
    n_chunks = per_w // _CHUNK
    nd = D // _L
    ng = (_CHUNK * K) // 128
    wid = lax.axis_index("s") * _NC + lax.axis_index("c")

    def chunk(it, carry):
        base = wid * per_w + it * _CHUNK
        pltpu.sync_copy(t_hbm.at[pl.ds(base, _CHUNK)], t_idx)
        pltpu.sync_copy(c_hbm.at[pl.ds(base, _CHUNK)], c_idx)
        pltpu.sync_copy(n_hbm.at[pl.ds(base * K, _CHUNK * K)], n_idx)
        cps = [
            pltpu.async_copy(iw_hbm.at[t_idx], v_rows, sem),
            pltpu.async_copy(ow_hbm.at[c_idx], up_rows, sem),
        ]
        for g in range(ng):
            cps.append(pltpu.async_copy(
                ow_hbm.at[n_idx.at[pl.ds(g * 128, 128)]],
                un_rows.at[pl.ds(g * 128, 128)], sem))
        for cp in cps:
            cp.wait()

        def elem(b, carry2):
            v = [v_rows[b, pl.ds(j * _L, _L)] for j in range(nd)]
            u = [up_rows[b, pl.ds(j * _L, _L)] for j in range(nd)]
            acc = v[0] * u[0]
            for j in range(1, nd):
                acc = acc + v[j] * u[j]
            pos_part[pl.ds(b * _L, _L)] = acc
            for k in range(K):
                w = [un_rows[b * K + k, pl.ds(j * _L, _L)] for j in range(nd)]
                nacc = v[0] * w[0]
                for j in range(1, nd):
                    nacc = nacc + v[j] * w[j]
                neg_part[pl.ds((b * K + k) * _L, _L)] = nacc
            return carry2

        lax.fori_loop(0, _CHUNK, elem, 0)
        pltpu.sync_copy(pos_part, pos_out.at[pl.ds(base * _L, _CHUNK * _L)])
        pltpu.sync_copy(neg_part,
                        neg_out.at[pl.ds(base * K * _L, _CHUNK * K * _L)])
        return carry

    lax.fori_loop(0, n_chunks, chunk, 0)


@functools.lru_cache(maxsize=None)
def _make_sc(B, K, D):
    mesh = plsc.VectorSubcoreMesh(core_axis_name="c", subcore_axis_name="s")
    return pl.kernel(
        functools.partial(_sc_body, K, D),
        out_type=[
            jax.ShapeDtypeStruct((B * _L,), jnp.float32),
            jax.ShapeDtypeStruct((B * K * _L,), jnp.float32),
        ],
        mesh=mesh,
        compiler_params=pltpu.CompilerParams(use_tc_tiling_on_sc=False),
        scratch_types=[
            pltpu.VMEM((_CHUNK,), jnp.int32),
            pltpu.VMEM((_CHUNK,), jnp.int32),
            pltpu.VMEM((_CHUNK * K,), jnp.int32),
            pltpu.VMEM((_CHUNK, D), jnp.float32),
            pltpu.VMEM((_CHUNK, D), jnp.float32),
            pltpu.VMEM((_CHUNK * K, D), jnp.float32),
            pltpu.VMEM((_CHUNK * _L,), jnp.float32),
            pltpu.VMEM((_CHUNK * K * _L,), jnp.float32),
            pltpu.SemaphoreType.DMA,
        ],
    )


def _loss_body(B, pos_ref, neg_ref, out_ref):
    i = pl.program_id(0)

    @pl.when(i == 0)
    def _init():
        out_ref[0, 0] = 0.0

    col = lax.broadcasted_iota(jnp.int32, (128, 128), 0) // _L
    row = lax.broadcasted_iota(jnp.int32, (128, 128), 1)
    sel = jnp.where((col == row) & (row < 128 // _L), 1.0, 0.0)
    valid = lax.broadcasted_iota(jnp.int32, (1, 128), 1) < 128 // _L

    def logsig(x):
        return jnp.minimum(x, 0.0) - jnp.log1p(jnp.exp(-jnp.abs(x)))

    ps = jnp.dot(pos_ref[:], sel, preferred_element_type=jnp.float32)
    ns = jnp.dot(neg_ref[:], sel, preferred_element_type=jnp.float32)
    tot = (jnp.sum(jnp.where(valid, logsig(ps), 0.0))
           + jnp.sum(jnp.where(valid, logsig(-ns), 0.0)))
    out_ref[0, 0] += -tot / B


def _tc_loss(pos2d, neg2d, B):
    n_steps = 8
    pb = pos2d.shape[0] // n_steps
    nb = neg2d.shape[0] // n_steps
    out = pl.pallas_call(
        functools.partial(_loss_body, B),
        grid=(n_steps,),
        in_specs=[
            pl.BlockSpec((pb, 128), lambda i: (i, 0)),
            pl.BlockSpec((nb, 128), lambda i: (i, 0)),
        ],
        out_specs=pl.BlockSpec(memory_space=pltpu.SMEM),
        out_shape=jax.ShapeDtypeStruct((1, 1), jnp.float32),
    )(pos2d, neg2d)
    return out[0, 0]


def kernel(targets, contexts, negatives, input_w, output_w):
    B = targets.shape[0]
    K = negatives.shape[1]
    D = input_w.shape[1]
    neg_flat = negatives.reshape(B * K)
    pos_part, neg_part = _make_sc(B, K, D)(
        targets, contexts, neg_flat, input_w, output_w)
    pos2d = pos_part.reshape(-1, 128)
    neg2d = neg_part.reshape(-1, 128)
    return _tc_loss(pos2d, neg2d, B)
